# baseline (speedup 1.0000x reference)
import functools

import jax
import jax.numpy as jnp
from jax import lax
from jax.experimental import pallas as pl
from jax.experimental.pallas import tpu as pltpu
from jax.experimental.pallas import tpu_sc as plsc

N = 10000
E = 320000
D = 128
G = 64

NC = 2
NS = 16
NW = NC * NS
EPW = E // NW
K = 80
NCH = EPW // K
RPT = 624
RPT_LAST = N - 15 * RPT

_mesh = plsc.VectorSubcoreMesh(core_axis_name="c", subcore_axis_name="s")


NSD = 5
assert NCH % NSD == 0
_deg_scratch = (
    [pltpu.VMEM((K, D), jnp.float32),
     pltpu.VMEM((16, D), jnp.float32)]
    + [pltpu.VMEM((K,), jnp.int32) for _ in range(NSD)]
    + [pltpu.SemaphoreType.DMA for _ in range(2 * NSD + 1)]
    + [pltpu.VMEM_SHARED((N, D), jnp.float32)]
)


@functools.partial(
    pl.kernel,
    mesh=_mesh,
    out_type=jax.ShapeDtypeStruct((NC, N, D), jnp.float32),
    scratch_types=_deg_scratch,
)
def _sc_degree(edge_hbm, ones_hbm, zeros_hbm, out_hbm, ones_v, zero_v,
               *rest):
    ic = rest[:NSD]
    semi = rest[NSD:2 * NSD]
    sems = rest[2 * NSD:3 * NSD]
    semz = rest[3 * NSD]
    acc = rest[3 * NSD + 1]

    c = lax.axis_index("c")
    s = lax.axis_index("s")
    base = (c * NS + s) * EPW
    start = s * RPT
    nz16 = lax.select(s == NS - 1, RPT_LAST // 16, RPT // 16)

    pltpu.sync_copy(ones_hbm, ones_v)
    pltpu.sync_copy(zeros_hbm, zero_v)

    def zfire(j, _):
        pltpu.async_copy(zero_v, acc.at[pl.ds(start + 16 * j, 16), :], semz)
        return 0

    def zdrain(j, _):
        pltpu.make_async_copy(zero_v, acc.at[pl.ds(start, 16), :],
                              semz).wait()
        return 0

    lax.fori_loop(0, nz16, zfire, 0)
    lax.fori_loop(0, nz16, zdrain, 0)
    plsc.subcore_barrier()

    def fire_idx(j, b):
        off = pl.multiple_of(E + base + j * K, 8)
        pltpu.async_copy(edge_hbm.at[pl.ds(off, K)], ic[b], semi[b])

    def wait_idx(b):
        off = pl.multiple_of(E + base, 8)
        pltpu.make_async_copy(edge_hbm.at[pl.ds(off, K)], ic[b],
                              semi[b]).wait()

    def fire_scatter(b):
        pltpu.async_copy(ones_v, acc.at[ic[b]], sems[b], add=True)

    def wait_scatter(b):
        pltpu.make_async_copy(ones_v, acc.at[ic[b]], sems[b]).wait()

    for b in range(NSD):
        fire_idx(b, b)
    for b in range(NSD):
        wait_idx(b)
        fire_scatter(b)

    def wave(w, _):
        for b in range(NSD):
            wait_scatter(b)
            fire_idx(NSD * w + b, b)
        for b in range(NSD):
            wait_idx(b)
            fire_scatter(b)
        return 0

    lax.fori_loop(1, NCH // NSD, wave, 0)
    for b in range(NSD):
        wait_scatter(b)

    plsc.subcore_barrier()

    @pl.when(s < NS - 1)
    def _():
        pltpu.sync_copy(acc.at[pl.ds(start, RPT), :],
                        out_hbm.at[c, pl.ds(start, RPT), :])

    @pl.when(s == NS - 1)
    def _():
        pltpu.sync_copy(acc.at[pl.ds(start, RPT_LAST), :],
                        out_hbm.at[c, pl.ds(start, RPT_LAST), :])


NSL = 3
_scatter_scratch = (
    [pltpu.VMEM((K, D), jnp.float32) for _ in range(NSL)]
    + [pltpu.VMEM((16, D), jnp.float32)]
    + [pltpu.VMEM((EPW,), jnp.int32)]
    + [pltpu.VMEM((K,), jnp.int32) for _ in range(NSL)]
    + [pltpu.SemaphoreType.DMA for _ in range(3 * NSL + 1)]
    + [pltpu.VMEM_SHARED((N, D), jnp.float32)]
)
_NWAVE = NCH // NSL
_NEPI = NCH - _NWAVE * NSL


@functools.partial(
    pl.kernel,
    mesh=_mesh,
    out_type=jax.ShapeDtypeStruct((NC, N, D), jnp.float32),
    scratch_types=_scatter_scratch,
)
def _sc_scatter(edge_hbm, zeros_hbm, s_hbm, out_hbm, *rest):
    rows = rest[:NSL]
    zbuf = rest[NSL]
    iall = rest[NSL + 1]
    ic = rest[NSL + 2:2 * NSL + 2]
    semi = rest[2 * NSL + 2:3 * NSL + 2]
    semg = rest[3 * NSL + 2:4 * NSL + 2]
    sems = rest[4 * NSL + 2:5 * NSL + 2]
    semz = rest[5 * NSL + 2]
    acc = rest[5 * NSL + 3]

    c = lax.axis_index("c")
    s = lax.axis_index("s")
    base = (c * NS + s) * EPW
    start = s * RPT
    nz16 = lax.select(s == NS - 1, RPT_LAST // 16, RPT // 16)

    pltpu.sync_copy(edge_hbm.at[pl.ds(pl.multiple_of(base, 8), EPW)], iall)
    pltpu.sync_copy(zeros_hbm, zbuf)

    def zfire(j, _):
        pltpu.async_copy(zbuf, acc.at[pl.ds(start + 16 * j, 16), :], semz)
        return 0

    def zdrain(j, _):
        pltpu.make_async_copy(zbuf, acc.at[pl.ds(start, 16), :],
                              semz).wait()
        return 0

    lax.fori_loop(0, nz16, zfire, 0)
    lax.fori_loop(0, nz16, zdrain, 0)
    plsc.subcore_barrier()

    def fire_gather(j, b):
        off = pl.multiple_of(j * K, 16)
        pltpu.async_copy(s_hbm.at[iall.at[pl.ds(off, K)]], rows[b], semg[b])

    def wait_gather(b):
        pltpu.make_async_copy(s_hbm.at[iall.at[pl.ds(0, K)]], rows[b],
                              semg[b]).wait()

    def fire_idx(j, b):
        off = pl.multiple_of(E + base + j * K, 8)
        pltpu.async_copy(edge_hbm.at[pl.ds(off, K)], ic[b], semi[b])

    def wait_idx(b):
        off = pl.multiple_of(E + base, 8)
        pltpu.make_async_copy(edge_hbm.at[pl.ds(off, K)], ic[b],
                              semi[b]).wait()

    def fire_scatter(b):
        pltpu.async_copy(rows[b], acc.at[ic[b]], sems[b], add=True)

    def wait_scatter(b):
        pltpu.make_async_copy(rows[b], acc.at[ic[b]], sems[b]).wait()

    for b in range(NSL):
        fire_gather(b, b)
        fire_idx(b, b)
    for b in range(NSL):
        wait_gather(b)
        wait_idx(b)
        fire_scatter(b)

    def wave(w, _):
        for b in range(NSL):
            wait_scatter(b)
            fire_gather(NSL * w + b, b)
            fire_idx(NSL * w + b, b)
        for b in range(NSL):
            wait_gather(b)
            wait_idx(b)
            fire_scatter(b)
        return 0

    lax.fori_loop(1, _NWAVE, wave, 0)

    for e in range(_NEPI):
        j = _NWAVE * NSL + e
        wait_scatter(e)
        fire_gather(j, e)
        fire_idx(j, e)
        wait_gather(e)
        wait_idx(e)
        fire_scatter(e)
    for b in range(NSL):
        wait_scatter(b)

    plsc.subcore_barrier()

    @pl.when(s < NS - 1)
    def _():
        pltpu.sync_copy(acc.at[pl.ds(start, RPT), :],
                        out_hbm.at[c, pl.ds(start, RPT), :])

    @pl.when(s == NS - 1)
    def _():
        pltpu.sync_copy(acc.at[pl.ds(start, RPT_LAST), :],
                        out_hbm.at[c, pl.ds(start, RPT_LAST), :])


RB = 1000
NRB = N // RB


def _dinv_block(d0, d1):
    deg = d0[:, 0:1] + d1[:, 0:1] + 1.0
    return lax.rsqrt(deg)


def _tc1_body(x_ref, w_ref, d0_ref, d1_ref, o_ref):
    dinv = _dinv_block(d0_ref[...], d1_ref[...])
    o_ref[...] = jnp.dot(x_ref[...], w_ref[...],
                         preferred_element_type=jnp.float32) * dinv


def _tc_mid_body(ga_ref, gb_ref, sp_ref, d0_ref, d1_ref, b_ref, w_ref, o_ref):
    dinv = _dinv_block(d0_ref[...], d1_ref[...])
    h = dinv * (ga_ref[...] + gb_ref[...] + sp_ref[...]) + b_ref[...]
    o_ref[...] = jnp.dot(h, w_ref[...],
                         preferred_element_type=jnp.float32) * dinv


def _tc_pool_body(ga_ref, gb_ref, sp_ref, d0_ref, d1_ref, b_ref, batch_ref,
                  o_ref, sum_s, cnt_s):
    i = pl.program_id(0)

    @pl.when(i == 0)
    def _():
        sum_s[...] = jnp.zeros((G, D), jnp.float32)
        cnt_s[...] = jnp.zeros((G, D), jnp.float32)

    dinv = _dinv_block(d0_ref[...], d1_ref[...])
    h3 = dinv * (ga_ref[...] + gb_ref[...] + sp_ref[...]) + b_ref[...]
    b = jnp.reshape(batch_ref[...], (RB, 1))
    onehot = (b == lax.broadcasted_iota(jnp.int32, (RB, G), 1)
              ).astype(jnp.float32)
    sum_s[...] += lax.dot_general(onehot, h3, (((0,), (0,)), ((), ())),
                                  preferred_element_type=jnp.float32)
    cnt_s[...] += lax.dot_general(onehot, jnp.ones((RB, D), jnp.float32),
                                  (((0,), (0,)), ((), ())),
                                  preferred_element_type=jnp.float32)

    @pl.when(i == NRB - 1)
    def _():
        o_ref[...] = sum_s[...] / jnp.maximum(cnt_s[...], 1.0)


_row_spec = pl.BlockSpec((RB, D), lambda i: (i, 0))
_deg_spec = pl.BlockSpec((RB, D), lambda i: (i, 0))
_w_spec = pl.BlockSpec((D, D), lambda i: (0, 0))
_b_spec = pl.BlockSpec((1, D), lambda i: (0, 0))

_tc1 = pl.pallas_call(
    _tc1_body,
    grid=(NRB,),
    in_specs=[_row_spec, _w_spec, _deg_spec, _deg_spec],
    out_specs=_row_spec,
    out_shape=jax.ShapeDtypeStruct((N, D), jnp.float32),
)

_tc_mid = pl.pallas_call(
    _tc_mid_body,
    grid=(NRB,),
    in_specs=[_row_spec, _row_spec, _row_spec, _deg_spec, _deg_spec,
              _b_spec, _w_spec],
    out_specs=_row_spec,
    out_shape=jax.ShapeDtypeStruct((N, D), jnp.float32),
)

_tc_pool = pl.pallas_call(
    _tc_pool_body,
    grid=(NRB,),
    in_specs=[_row_spec, _row_spec, _row_spec, _deg_spec, _deg_spec, _b_spec,
              pl.BlockSpec((1, 1, RB), lambda i: (i, 0, 0))],
    out_specs=pl.BlockSpec((G, D), lambda i: (0, 0)),
    out_shape=jax.ShapeDtypeStruct((G, D), jnp.float32),
    scratch_shapes=[pltpu.VMEM((G, D), jnp.float32),
                    pltpu.VMEM((G, D), jnp.float32)],
)


def kernel(x, edge_index, batch, W1, b1, W2, b2, W3, b3):
    edge_flat = edge_index.reshape(-1)
    zrows = jnp.zeros((16, D), jnp.float32)
    orows = jnp.ones((K, D), jnp.float32)

    degp = _sc_degree(edge_flat, orows, zrows)
    d0, d1 = degp[0], degp[1]

    s1 = _tc1(x, W1, d0, d1)
    g1 = _sc_scatter(edge_flat, zrows, s1)
    s2 = _tc_mid(g1[0], g1[1], s1, d0, d1, b1.reshape(1, D), W2)
    g2 = _sc_scatter(edge_flat, zrows, s2)
    s3 = _tc_mid(g2[0], g2[1], s2, d0, d1, b2.reshape(1, D), W3)
    g3 = _sc_scatter(edge_flat, zrows, s3)

    batch3 = batch.reshape(NRB, 1, RB)
    return _tc_pool(g3[0], g3[1], s3, d0, d1, b3.reshape(1, D), batch3)


# speedup vs baseline: 1.1396x; 1.1396x over previous
import functools

import jax
import jax.numpy as jnp
from jax import lax
from jax.experimental import pallas as pl
from jax.experimental.pallas import tpu as pltpu
from jax.experimental.pallas import tpu_sc as plsc

N = 10000
E = 320000
D = 128
G = 64

NC = 2
NS = 16
NW = NC * NS
EPW = E // NW
K = 80
NCH = EPW // K
RPT = 624
RPT_LAST = N - 15 * RPT

_mesh = plsc.VectorSubcoreMesh(core_axis_name="c", subcore_axis_name="s")


NSD = 5
assert NCH % NSD == 0
_deg_scratch = (
    [pltpu.VMEM((K, D), jnp.float32),
     pltpu.VMEM((16, D), jnp.float32)]
    + [pltpu.VMEM((K,), jnp.int32) for _ in range(NSD)]
    + [pltpu.SemaphoreType.DMA for _ in range(2 * NSD + 1)]
    + [pltpu.VMEM_SHARED((N, D), jnp.float32)]
)


@functools.partial(
    pl.kernel,
    mesh=_mesh,
    out_type=jax.ShapeDtypeStruct((NC, N, D), jnp.float32),
    scratch_types=_deg_scratch,
)
def _sc_degree(edge_hbm, ones_hbm, zeros_hbm, out_hbm, ones_v, zero_v,
               *rest):
    ic = rest[:NSD]
    semi = rest[NSD:2 * NSD]
    sems = rest[2 * NSD:3 * NSD]
    semz = rest[3 * NSD]
    acc = rest[3 * NSD + 1]

    c = lax.axis_index("c")
    s = lax.axis_index("s")
    base = (c * NS + s) * EPW
    start = s * RPT
    nz16 = lax.select(s == NS - 1, RPT_LAST // 16, RPT // 16)

    pltpu.sync_copy(ones_hbm, ones_v)
    pltpu.sync_copy(zeros_hbm, zero_v)

    def zfire(j, _):
        pltpu.async_copy(zero_v, acc.at[pl.ds(start + 16 * j, 16), :], semz)
        return 0

    def zdrain(j, _):
        pltpu.make_async_copy(zero_v, acc.at[pl.ds(start, 16), :],
                              semz).wait()
        return 0

    lax.fori_loop(0, nz16, zfire, 0)
    lax.fori_loop(0, nz16, zdrain, 0)
    plsc.subcore_barrier()

    def fire_idx(j, b):
        off = pl.multiple_of(E + base + j * K, 8)
        pltpu.async_copy(edge_hbm.at[pl.ds(off, K)], ic[b], semi[b])

    def wait_idx(b):
        off = pl.multiple_of(E + base, 8)
        pltpu.make_async_copy(edge_hbm.at[pl.ds(off, K)], ic[b],
                              semi[b]).wait()

    def fire_scatter(b):
        pltpu.async_copy(ones_v, acc.at[ic[b]], sems[b], add=True)

    def wait_scatter(b):
        pltpu.make_async_copy(ones_v, acc.at[ic[b]], sems[b]).wait()

    for b in range(NSD):
        fire_idx(b, b)
    for b in range(NSD):
        wait_idx(b)
        fire_scatter(b)

    def wave(w, _):
        for b in range(NSD):
            wait_scatter(b)
            fire_idx(NSD * w + b, b)
        for b in range(NSD):
            wait_idx(b)
            fire_scatter(b)
        return 0

    lax.fori_loop(1, NCH // NSD, wave, 0)
    for b in range(NSD):
        wait_scatter(b)

    plsc.subcore_barrier()

    @pl.when(s < NS - 1)
    def _():
        pltpu.sync_copy(acc.at[pl.ds(start, RPT), :],
                        out_hbm.at[c, pl.ds(start, RPT), :])

    @pl.when(s == NS - 1)
    def _():
        pltpu.sync_copy(acc.at[pl.ds(start, RPT_LAST), :],
                        out_hbm.at[c, pl.ds(start, RPT_LAST), :])


NSL = 3
_scatter_scratch = (
    [pltpu.VMEM((K, D), jnp.float32) for _ in range(NSL)]
    + [pltpu.VMEM((16, D), jnp.float32)]
    + [pltpu.VMEM((EPW,), jnp.int32)]
    + [pltpu.VMEM((K,), jnp.int32) for _ in range(NSL)]
    + [pltpu.SemaphoreType.DMA for _ in range(3 * NSL + 1)]
    + [pltpu.VMEM_SHARED((N, D), jnp.float32)]
)
_NWAVE = NCH // NSL
_NEPI = NCH - _NWAVE * NSL


@functools.partial(
    pl.kernel,
    mesh=_mesh,
    out_type=jax.ShapeDtypeStruct((NC, N, D), jnp.float32),
    scratch_types=_scatter_scratch,
)
def _sc_scatter(edge_hbm, zeros_hbm, s_hbm, out_hbm, *rest):
    rows = rest[:NSL]
    zbuf = rest[NSL]
    iall = rest[NSL + 1]
    ic = rest[NSL + 2:2 * NSL + 2]
    semi = rest[2 * NSL + 2:3 * NSL + 2]
    semg = rest[3 * NSL + 2:4 * NSL + 2]
    sems = rest[4 * NSL + 2:5 * NSL + 2]
    semz = rest[5 * NSL + 2]
    acc = rest[5 * NSL + 3]

    c = lax.axis_index("c")
    s = lax.axis_index("s")
    base = (c * NS + s) * EPW
    start = s * RPT
    nz16 = lax.select(s == NS - 1, RPT_LAST // 16, RPT // 16)

    pltpu.sync_copy(edge_hbm.at[pl.ds(pl.multiple_of(base, 8), EPW)], iall)
    pltpu.sync_copy(zeros_hbm, zbuf)

    def zfire(j, _):
        pltpu.async_copy(zbuf, acc.at[pl.ds(start + 16 * j, 16), :], semz)
        return 0

    def zdrain(j, _):
        pltpu.make_async_copy(zbuf, acc.at[pl.ds(start, 16), :],
                              semz).wait()
        return 0

    lax.fori_loop(0, nz16, zfire, 0)
    lax.fori_loop(0, nz16, zdrain, 0)
    plsc.subcore_barrier()

    def fire_gather(j, b):
        off = pl.multiple_of(j * K, 16)
        pltpu.async_copy(s_hbm.at[iall.at[pl.ds(off, K)]], rows[b], semg[b])

    def wait_gather(b):
        pltpu.make_async_copy(s_hbm.at[iall.at[pl.ds(0, K)]], rows[b],
                              semg[b]).wait()

    def fire_idx(j, b):
        off = pl.multiple_of(E + base + j * K, 8)
        pltpu.async_copy(edge_hbm.at[pl.ds(off, K)], ic[b], semi[b])

    def wait_idx(b):
        off = pl.multiple_of(E + base, 8)
        pltpu.make_async_copy(edge_hbm.at[pl.ds(off, K)], ic[b],
                              semi[b]).wait()

    def fire_scatter(b):
        pltpu.async_copy(rows[b], acc.at[ic[b]], sems[b], add=True)

    def wait_scatter(b):
        pltpu.make_async_copy(rows[b], acc.at[ic[b]], sems[b]).wait()

    def commit(bb):
        wait_gather(bb)
        wait_idx(bb)
        fire_scatter(bb)

    for b in range(NSL):
        fire_gather(b, b)
        fire_idx(b, b)
    for b in range(NSL - 1):
        commit(b)

    def wave(w, _):
        for b in range(NSL):
            wait_scatter(b)
            fire_gather(NSL * w + b, b)
            fire_idx(NSL * w + b, b)
            commit((b + NSL - 1) % NSL)
        return 0

    lax.fori_loop(1, _NWAVE, wave, 0)

    for e in range(_NEPI):
        j = _NWAVE * NSL + e
        wait_scatter(e)
        fire_gather(j, e)
        fire_idx(j, e)
        commit((e + NSL - 1) % NSL)
    commit((_NEPI + NSL - 1) % NSL)
    for b in range(NSL):
        wait_scatter(b)

    plsc.subcore_barrier()

    @pl.when(s < NS - 1)
    def _():
        pltpu.sync_copy(acc.at[pl.ds(start, RPT), :],
                        out_hbm.at[c, pl.ds(start, RPT), :])

    @pl.when(s == NS - 1)
    def _():
        pltpu.sync_copy(acc.at[pl.ds(start, RPT_LAST), :],
                        out_hbm.at[c, pl.ds(start, RPT_LAST), :])


RB = 1000
NRB = N // RB


def _dinv_block(d0, d1):
    deg = d0[:, 0:1] + d1[:, 0:1] + 1.0
    return lax.rsqrt(deg)


def _tc1_body(x_ref, w_ref, d0_ref, d1_ref, o_ref):
    dinv = _dinv_block(d0_ref[...], d1_ref[...])
    o_ref[...] = jnp.dot(x_ref[...], w_ref[...],
                         preferred_element_type=jnp.float32) * dinv


def _tc_mid_body(ga_ref, gb_ref, sp_ref, d0_ref, d1_ref, b_ref, w_ref, o_ref):
    dinv = _dinv_block(d0_ref[...], d1_ref[...])
    h = dinv * (ga_ref[...] + gb_ref[...] + sp_ref[...]) + b_ref[...]
    o_ref[...] = jnp.dot(h, w_ref[...],
                         preferred_element_type=jnp.float32) * dinv


def _tc_pool_body(ga_ref, gb_ref, sp_ref, d0_ref, d1_ref, b_ref, batch_ref,
                  o_ref, sum_s, cnt_s):
    i = pl.program_id(0)

    @pl.when(i == 0)
    def _():
        sum_s[...] = jnp.zeros((G, D), jnp.float32)
        cnt_s[...] = jnp.zeros((G, D), jnp.float32)

    dinv = _dinv_block(d0_ref[...], d1_ref[...])
    h3 = dinv * (ga_ref[...] + gb_ref[...] + sp_ref[...]) + b_ref[...]
    b = jnp.reshape(batch_ref[...], (RB, 1))
    onehot = (b == lax.broadcasted_iota(jnp.int32, (RB, G), 1)
              ).astype(jnp.float32)
    sum_s[...] += lax.dot_general(onehot, h3, (((0,), (0,)), ((), ())),
                                  preferred_element_type=jnp.float32)
    cnt_s[...] += lax.dot_general(onehot, jnp.ones((RB, D), jnp.float32),
                                  (((0,), (0,)), ((), ())),
                                  preferred_element_type=jnp.float32)

    @pl.when(i == NRB - 1)
    def _():
        o_ref[...] = sum_s[...] / jnp.maximum(cnt_s[...], 1.0)


_row_spec = pl.BlockSpec((RB, D), lambda i: (i, 0))
_deg_spec = pl.BlockSpec((RB, D), lambda i: (i, 0))
_w_spec = pl.BlockSpec((D, D), lambda i: (0, 0))
_b_spec = pl.BlockSpec((1, D), lambda i: (0, 0))

_tc1 = pl.pallas_call(
    _tc1_body,
    grid=(NRB,),
    in_specs=[_row_spec, _w_spec, _deg_spec, _deg_spec],
    out_specs=_row_spec,
    out_shape=jax.ShapeDtypeStruct((N, D), jnp.float32),
)

_tc_mid = pl.pallas_call(
    _tc_mid_body,
    grid=(NRB,),
    in_specs=[_row_spec, _row_spec, _row_spec, _deg_spec, _deg_spec,
              _b_spec, _w_spec],
    out_specs=_row_spec,
    out_shape=jax.ShapeDtypeStruct((N, D), jnp.float32),
)

_tc_pool = pl.pallas_call(
    _tc_pool_body,
    grid=(NRB,),
    in_specs=[_row_spec, _row_spec, _row_spec, _deg_spec, _deg_spec, _b_spec,
              pl.BlockSpec((1, 1, RB), lambda i: (i, 0, 0))],
    out_specs=pl.BlockSpec((G, D), lambda i: (0, 0)),
    out_shape=jax.ShapeDtypeStruct((G, D), jnp.float32),
    scratch_shapes=[pltpu.VMEM((G, D), jnp.float32),
                    pltpu.VMEM((G, D), jnp.float32)],
)


def kernel(x, edge_index, batch, W1, b1, W2, b2, W3, b3):
    edge_flat = edge_index.reshape(-1)
    zrows = jnp.zeros((16, D), jnp.float32)
    orows = jnp.ones((K, D), jnp.float32)

    degp = _sc_degree(edge_flat, orows, zrows)
    d0, d1 = degp[0], degp[1]

    s1 = _tc1(x, W1, d0, d1)
    g1 = _sc_scatter(edge_flat, zrows, s1)
    s2 = _tc_mid(g1[0], g1[1], s1, d0, d1, b1.reshape(1, D), W2)
    g2 = _sc_scatter(edge_flat, zrows, s2)
    s3 = _tc_mid(g2[0], g2[1], s2, d0, d1, b2.reshape(1, D), W3)
    g3 = _sc_scatter(edge_flat, zrows, s3)

    batch3 = batch.reshape(NRB, 1, RB)
    return _tc_pool(g3[0], g3[1], s3, d0, d1, b3.reshape(1, D), batch3)


# speedup vs baseline: 1.1419x; 1.0020x over previous
import functools

import jax
import jax.numpy as jnp
from jax import lax
from jax.experimental import pallas as pl
from jax.experimental.pallas import tpu as pltpu
from jax.experimental.pallas import tpu_sc as plsc

N = 10000
E = 320000
D = 128
G = 64

NC = 2
NS = 16
NW = NC * NS
EPW = E // NW
K = 80
NCH = EPW // K
RPT = 624
RPT_LAST = N - 15 * RPT

_mesh = plsc.VectorSubcoreMesh(core_axis_name="c", subcore_axis_name="s")


NSD = 5
assert NCH % NSD == 0
_deg_scratch = (
    [pltpu.VMEM((K, D), jnp.float32),
     pltpu.VMEM((16, D), jnp.float32)]
    + [pltpu.VMEM((K,), jnp.int32) for _ in range(NSD)]
    + [pltpu.SemaphoreType.DMA for _ in range(2 * NSD + 1)]
    + [pltpu.VMEM_SHARED((N, D), jnp.float32)]
)


@functools.partial(
    pl.kernel,
    mesh=_mesh,
    out_type=jax.ShapeDtypeStruct((NC, N, D), jnp.float32),
    scratch_types=_deg_scratch,
)
def _sc_degree(edge_hbm, ones_hbm, zeros_hbm, out_hbm, ones_v, zero_v,
               *rest):
    ic = rest[:NSD]
    semi = rest[NSD:2 * NSD]
    sems = rest[2 * NSD:3 * NSD]
    semz = rest[3 * NSD]
    acc = rest[3 * NSD + 1]

    c = lax.axis_index("c")
    s = lax.axis_index("s")
    base = (c * NS + s) * EPW
    start = s * RPT
    nz16 = lax.select(s == NS - 1, RPT_LAST // 16, RPT // 16)

    pltpu.sync_copy(ones_hbm, ones_v)
    pltpu.sync_copy(zeros_hbm, zero_v)

    def zfire(j, _):
        pltpu.async_copy(zero_v, acc.at[pl.ds(start + 16 * j, 16), :], semz)
        return 0

    def zdrain(j, _):
        pltpu.make_async_copy(zero_v, acc.at[pl.ds(start, 16), :],
                              semz).wait()
        return 0

    lax.fori_loop(0, nz16, zfire, 0)
    lax.fori_loop(0, nz16, zdrain, 0)
    plsc.subcore_barrier()

    def fire_idx(j, b):
        off = pl.multiple_of(E + base + j * K, 8)
        pltpu.async_copy(edge_hbm.at[pl.ds(off, K)], ic[b], semi[b])

    def wait_idx(b):
        off = pl.multiple_of(E + base, 8)
        pltpu.make_async_copy(edge_hbm.at[pl.ds(off, K)], ic[b],
                              semi[b]).wait()

    def fire_scatter(b):
        pltpu.async_copy(ones_v, acc.at[ic[b]], sems[b], add=True)

    def wait_scatter(b):
        pltpu.make_async_copy(ones_v, acc.at[ic[b]], sems[b]).wait()

    def commit(bb):
        wait_idx(bb)
        fire_scatter(bb)

    for b in range(NSD):
        fire_idx(b, b)
    for b in range(NSD - 1):
        commit(b)

    def wave(w, _):
        for b in range(NSD):
            wait_scatter(b)
            fire_idx(NSD * w + b, b)
            commit((b + NSD - 1) % NSD)
        return 0

    lax.fori_loop(1, NCH // NSD, wave, 0)
    commit(NSD - 1)
    for b in range(NSD):
        wait_scatter(b)

    plsc.subcore_barrier()

    @pl.when(s < NS - 1)
    def _():
        pltpu.sync_copy(acc.at[pl.ds(start, RPT), :],
                        out_hbm.at[c, pl.ds(start, RPT), :])

    @pl.when(s == NS - 1)
    def _():
        pltpu.sync_copy(acc.at[pl.ds(start, RPT_LAST), :],
                        out_hbm.at[c, pl.ds(start, RPT_LAST), :])


NSL = 3
_scatter_scratch = (
    [pltpu.VMEM((K, D), jnp.float32) for _ in range(NSL)]
    + [pltpu.VMEM((16, D), jnp.float32)]
    + [pltpu.VMEM((EPW,), jnp.int32)]
    + [pltpu.VMEM((K,), jnp.int32) for _ in range(NSL)]
    + [pltpu.SemaphoreType.DMA for _ in range(3 * NSL + 1)]
    + [pltpu.VMEM_SHARED((N, D), jnp.float32)]
)
_NWAVE = NCH // NSL
_NEPI = NCH - _NWAVE * NSL


@functools.partial(
    pl.kernel,
    mesh=_mesh,
    out_type=jax.ShapeDtypeStruct((NC, N, D), jnp.float32),
    scratch_types=_scatter_scratch,
)
def _sc_scatter(edge_hbm, zeros_hbm, s_hbm, out_hbm, *rest):
    rows = rest[:NSL]
    zbuf = rest[NSL]
    iall = rest[NSL + 1]
    ic = rest[NSL + 2:2 * NSL + 2]
    semi = rest[2 * NSL + 2:3 * NSL + 2]
    semg = rest[3 * NSL + 2:4 * NSL + 2]
    sems = rest[4 * NSL + 2:5 * NSL + 2]
    semz = rest[5 * NSL + 2]
    acc = rest[5 * NSL + 3]

    c = lax.axis_index("c")
    s = lax.axis_index("s")
    base = (c * NS + s) * EPW
    start = s * RPT
    nz16 = lax.select(s == NS - 1, RPT_LAST // 16, RPT // 16)

    pltpu.sync_copy(edge_hbm.at[pl.ds(pl.multiple_of(base, 8), EPW)], iall)
    pltpu.sync_copy(zeros_hbm, zbuf)

    def zfire(j, _):
        pltpu.async_copy(zbuf, acc.at[pl.ds(start + 16 * j, 16), :], semz)
        return 0

    def zdrain(j, _):
        pltpu.make_async_copy(zbuf, acc.at[pl.ds(start, 16), :],
                              semz).wait()
        return 0

    lax.fori_loop(0, nz16, zfire, 0)
    lax.fori_loop(0, nz16, zdrain, 0)
    plsc.subcore_barrier()

    def fire_gather(j, b):
        off = pl.multiple_of(j * K, 16)
        pltpu.async_copy(s_hbm.at[iall.at[pl.ds(off, K)]], rows[b], semg[b])

    def wait_gather(b):
        pltpu.make_async_copy(s_hbm.at[iall.at[pl.ds(0, K)]], rows[b],
                              semg[b]).wait()

    def fire_idx(j, b):
        off = pl.multiple_of(E + base + j * K, 8)
        pltpu.async_copy(edge_hbm.at[pl.ds(off, K)], ic[b], semi[b])

    def wait_idx(b):
        off = pl.multiple_of(E + base, 8)
        pltpu.make_async_copy(edge_hbm.at[pl.ds(off, K)], ic[b],
                              semi[b]).wait()

    def fire_scatter(b):
        pltpu.async_copy(rows[b], acc.at[ic[b]], sems[b], add=True)

    def wait_scatter(b):
        pltpu.make_async_copy(rows[b], acc.at[ic[b]], sems[b]).wait()

    def commit(bb):
        wait_gather(bb)
        wait_idx(bb)
        fire_scatter(bb)

    for b in range(NSL):
        fire_gather(b, b)
        fire_idx(b, b)
    for b in range(NSL - 1):
        commit(b)

    def wave(w, _):
        for b in range(NSL):
            wait_scatter(b)
            fire_gather(NSL * w + b, b)
            fire_idx(NSL * w + b, b)
            commit((b + NSL - 1) % NSL)
        return 0

    lax.fori_loop(1, _NWAVE, wave, 0)

    for e in range(_NEPI):
        j = _NWAVE * NSL + e
        wait_scatter(e)
        fire_gather(j, e)
        fire_idx(j, e)
        commit((e + NSL - 1) % NSL)
    commit((_NEPI + NSL - 1) % NSL)
    for b in range(NSL):
        wait_scatter(b)

    plsc.subcore_barrier()

    @pl.when(s < NS - 1)
    def _():
        pltpu.sync_copy(acc.at[pl.ds(start, RPT), :],
                        out_hbm.at[c, pl.ds(start, RPT), :])

    @pl.when(s == NS - 1)
    def _():
        pltpu.sync_copy(acc.at[pl.ds(start, RPT_LAST), :],
                        out_hbm.at[c, pl.ds(start, RPT_LAST), :])


RB = 1000
NRB = N // RB


def _dinv_block(d0, d1):
    deg = (d0[:, 0:1].astype(jnp.float32) + d1[:, 0:1].astype(jnp.float32)
           + 1.0)
    return lax.rsqrt(deg)


def _tc1_body(x_ref, w_ref, d0_ref, d1_ref, o_ref):
    dinv = _dinv_block(d0_ref[...], d1_ref[...])
    o_ref[...] = jnp.dot(x_ref[...], w_ref[...],
                         preferred_element_type=jnp.float32) * dinv


def _tc_mid_body(ga_ref, gb_ref, sp_ref, d0_ref, d1_ref, b_ref, w_ref, o_ref):
    dinv = _dinv_block(d0_ref[...], d1_ref[...])
    h = dinv * (ga_ref[...] + gb_ref[...] + sp_ref[...]) + b_ref[...]
    o_ref[...] = jnp.dot(h, w_ref[...],
                         preferred_element_type=jnp.float32) * dinv


def _tc_pool_body(ga_ref, gb_ref, sp_ref, d0_ref, d1_ref, b_ref, batch_ref,
                  o_ref, sum_s, cnt_s):
    i = pl.program_id(0)

    @pl.when(i == 0)
    def _():
        sum_s[...] = jnp.zeros((G, D), jnp.float32)
        cnt_s[...] = jnp.zeros((G, D), jnp.float32)

    dinv = _dinv_block(d0_ref[...], d1_ref[...])
    h3 = dinv * (ga_ref[...] + gb_ref[...] + sp_ref[...]) + b_ref[...]
    b = jnp.reshape(batch_ref[...], (RB, 1))
    onehot = (b == lax.broadcasted_iota(jnp.int32, (RB, G), 1)
              ).astype(jnp.float32)
    sum_s[...] += lax.dot_general(onehot, h3, (((0,), (0,)), ((), ())),
                                  preferred_element_type=jnp.float32)
    cnt_s[...] += lax.dot_general(onehot, jnp.ones((RB, D), jnp.float32),
                                  (((0,), (0,)), ((), ())),
                                  preferred_element_type=jnp.float32)

    @pl.when(i == NRB - 1)
    def _():
        o_ref[...] = sum_s[...] / jnp.maximum(cnt_s[...], 1.0)


_row_spec = pl.BlockSpec((RB, D), lambda i: (i, 0))
_deg_spec = pl.BlockSpec((RB, D), lambda i: (i, 0))
_w_spec = pl.BlockSpec((D, D), lambda i: (0, 0))
_b_spec = pl.BlockSpec((1, D), lambda i: (0, 0))

_tc1 = pl.pallas_call(
    _tc1_body,
    grid=(NRB,),
    in_specs=[_row_spec, _w_spec, _deg_spec, _deg_spec],
    out_specs=_row_spec,
    out_shape=jax.ShapeDtypeStruct((N, D), jnp.float32),
)

_tc_mid = pl.pallas_call(
    _tc_mid_body,
    grid=(NRB,),
    in_specs=[_row_spec, _row_spec, _row_spec, _deg_spec, _deg_spec,
              _b_spec, _w_spec],
    out_specs=_row_spec,
    out_shape=jax.ShapeDtypeStruct((N, D), jnp.float32),
)

_tc_pool = pl.pallas_call(
    _tc_pool_body,
    grid=(NRB,),
    in_specs=[_row_spec, _row_spec, _row_spec, _deg_spec, _deg_spec, _b_spec,
              pl.BlockSpec((1, 1, RB), lambda i: (i, 0, 0))],
    out_specs=pl.BlockSpec((G, D), lambda i: (0, 0)),
    out_shape=jax.ShapeDtypeStruct((G, D), jnp.float32),
    scratch_shapes=[pltpu.VMEM((G, D), jnp.float32),
                    pltpu.VMEM((G, D), jnp.float32)],
)


def kernel(x, edge_index, batch, W1, b1, W2, b2, W3, b3):
    edge_flat = edge_index.reshape(-1)
    zrows = jnp.zeros((16, D), jnp.float32)

    degp = _sc_degree(edge_flat, jnp.ones((K, D), jnp.float32), zrows)
    d0, d1 = degp[0], degp[1]

    s1 = _tc1(x, W1, d0, d1)
    g1 = _sc_scatter(edge_flat, zrows, s1)
    s2 = _tc_mid(g1[0], g1[1], s1, d0, d1, b1.reshape(1, D), W2)
    g2 = _sc_scatter(edge_flat, zrows, s2)
    s3 = _tc_mid(g2[0], g2[1], s2, d0, d1, b2.reshape(1, D), W3)
    g3 = _sc_scatter(edge_flat, zrows, s3)

    batch3 = batch.reshape(NRB, 1, RB)
    return _tc_pool(g3[0], g3[1], s3, d0, d1, b3.reshape(1, D), batch3)
